# baseline (speedup 1.0000x reference)
import functools

import jax
import jax.numpy as jnp
from jax import lax
from jax.experimental import pallas as pl
from jax.experimental.pallas import tpu as pltpu
from jax.experimental.pallas import tpu_sc as plsc

HIDDEN = 64
SEQ = 200
BATCH = 4096
TOTAL = BATCH * SEQ
VOCAB = 1000000
NUM_WORKERS = 32
PER_WORKER = TOTAL // NUM_WORKERS
CHUNK = 128
NCHUNK = PER_WORKER // CHUNK
NBT = BATCH // 128

NVT = (VOCAB + 127) // 128
PAIR_ROWS = NVT * 64

_mesh = plsc.VectorSubcoreMesh(core_axis_name="c", subcore_axis_name="s")

def _iota16():
    return jax.lax.iota(jnp.int32, 16)


@functools.partial(
    pl.kernel,
    mesh=_mesh,
    out_type=jax.ShapeDtypeStruct((PAIR_ROWS, 128), jnp.float32),
    scratch_types=[
        pltpu.VMEM((64, 128), jnp.float32),
        pltpu.VMEM((64, 128), jnp.float32),
    ],
    compiler_params=pltpu.CompilerParams(needs_layout_passes=False),
)
def _transpose_kernel(table_hbm, pairs_hbm, tile_v, dst_v):
    wid = lax.axis_index("s") * 2 + lax.axis_index("c")
    lo = wid * NVT // NUM_WORKERS
    hi = (wid + 1) * NVT // NUM_WORKERS

    def body(t, carry):
        ncols = jnp.minimum(VOCAB - t * 128, 128)
        pltpu.sync_copy(
            table_hbm.at[pl.ds(0, 64), pl.ds(t * 128, 128)], tile_v)

        def prow(p, c2):
            @pl.when(2 * p < ncols)
            def _():
                for q in range(4):
                    hs = q * 16 + _iota16()
                    v0 = plsc.load_gather(tile_v, [hs, jnp.full((16,), 2 * p)])
                    dst_v[p, pl.ds(q * 16, 16)] = v0
                    v1 = plsc.load_gather(
                        tile_v, [hs, jnp.full((16,), 2 * p + 1)])
                    dst_v[p, pl.ds(64 + q * 16, 16)] = v1
            return c2

        lax.fori_loop(0, 64, prow, 0)
        pltpu.sync_copy(dst_v, pairs_hbm.at[pl.ds(t * 64, 64)])
        return carry

    lax.fori_loop(lo, hi, body, 0)


@functools.partial(
    pl.kernel,
    mesh=_mesh,
    out_type=jax.ShapeDtypeStruct((SEQ, 8, NBT, 8, 128), jnp.float32),
    scratch_types=[
        pltpu.VMEM((NCHUNK, CHUNK), jnp.int32),
        pltpu.VMEM((2, CHUNK), jnp.int32),
        pltpu.VMEM((2, CHUNK), jnp.int32),
        pltpu.VMEM((2, CHUNK, 128), jnp.float32),
        pltpu.VMEM((2, 8, 8, 128), jnp.float32),
        pltpu.SemaphoreType.DMA,
        pltpu.SemaphoreType.DMA,
    ],
    compiler_params=pltpu.CompilerParams(needs_layout_passes=False),
)
def _gather_kernel(idx_hbm, pairs_hbm, out_hbm,
                   idx_v, pidx_v, pcol_v, rows_v, trans_v, gsem, ssem):
    wid = lax.axis_index("s") * 2 + lax.axis_index("c")
    ubase = wid * NCHUNK

    pltpu.sync_copy(idx_hbm.at[wid], idx_v)

    def start_gather(i):
        b = i % 2
        for q in range(CHUNK // 16):
            v = idx_v[i, pl.ds(q * 16, 16)]
            pidx_v[b, pl.ds(q * 16, 16)] = v >> 1
            pcol_v[b, pl.ds(q * 16, 16)] = (v & 1) * 64
        pltpu.async_copy(pairs_hbm.at[pidx_v.at[b]], rows_v.at[b], gsem)

    def wait_gather():
        pltpu.make_async_copy(
            pairs_hbm.at[pidx_v.at[0]], rows_v.at[0], gsem).wait()

    def start_store(i):
        u = ubase + i
        pltpu.async_copy(
            trans_v.at[i % 2],
            out_hbm.at[u // NBT, pl.ds(0, 8), u % NBT],
            ssem)

    def wait_store():
        pltpu.make_async_copy(
            trans_v.at[0], out_hbm.at[0, pl.ds(0, 8), 0], ssem).wait()

    def transpose(i):
        b = i % 2
        rows = rows_v.at[b]

        def th(h, c2):
            for q in range(8):
                bs = q * 16 + _iota16()
                pc = pcol_v[b, pl.ds(q * 16, 16)]
                vals = plsc.load_gather(rows, [bs, pc + h])
                trans_v[b, h // 8, h % 8, pl.ds(q * 16, 16)] = vals
            return c2

        lax.fori_loop(0, 64, th, 0)

    start_gather(0)

    def body(i, carry):
        @pl.when(i >= 2)
        def _():
            wait_store()

        @pl.when(i + 1 < NCHUNK)
        def _():
            start_gather(i + 1)

        wait_gather()
        transpose(i)
        start_store(i)
        return carry

    lax.fori_loop(0, NCHUNK, body, 0)
    wait_store()
    wait_store()


def kernel(input_ids, word_table):
    idx_t = input_ids.T.reshape(NUM_WORKERS, NCHUNK, CHUNK).astype(jnp.int32)
    pairs = _transpose_kernel(word_table.T)
    out5 = _gather_kernel(idx_t, pairs)
    return out5.transpose(2, 4, 0, 1, 3).reshape(BATCH, SEQ, HIDDEN)
